# baseline (speedup 1.0000x reference)
import functools

import jax
import jax.numpy as jnp
from jax import lax
from jax.experimental import pallas as pl
from jax.experimental.pallas import tpu as pltpu
from jax.experimental.pallas import tpu_sc as plsc

_K = 512
_COLS = 32768
_ROWS = 128
_L = 16
_NCHUNK = _COLS // _L


def _sortable_key(xv):
    u = lax.bitcast_convert_type(xv, jnp.uint32)
    neg = u >> jnp.uint32(31)
    return u ^ (neg * jnp.uint32(0x7FFFFFFF) + jnp.uint32(0x80000000))


def _scan_hist(hist_ref, nbins, k_needed):
    nch = nbins // _L

    def body(i, carry):
        cum, found, b_sel, cgt_sel = carry
        j = nch - 1 - i
        chunk = hist_ref[pl.ds(j * _L, _L)]
        rchunk = lax.rev(chunk, (0,))
        rcs = jnp.cumsum(rchunk)
        rc = lax.rev(rcs, (0,))
        cge = rc + cum
        cond = cge >= k_needed
        npos = jnp.sum(cond.astype(jnp.int32), axis=0)
        has = npos > 0
        b_cand = _L * j + npos - 1
        big = jnp.int32(0x7FFFFFFF)
        cge_sel = jnp.min(jnp.where(cond, cge, big), axis=0)
        cgt_in = jnp.max(jnp.where(cond, jnp.int32(0), cge), axis=0)
        cgt_cand = jnp.where(npos == _L, cum, cgt_in)
        del cge_sel
        take = jnp.logical_and(jnp.logical_not(found), has)
        b_new = jnp.where(take, b_cand, b_sel)
        cgt_new = jnp.where(take, cgt_cand, cgt_sel)
        cum_new = cum + rcs[_L - 1]
        return cum_new, jnp.logical_or(found, has), b_new, cgt_new

    init = (jnp.int32(0), jnp.bool_(False), jnp.int32(0), jnp.int32(0))
    _, _, b_sel, cgt_sel = lax.fori_loop(0, nch, body, init)
    return b_sel, cgt_sel


def _zero_hist(hist_ref, nbins):
    def body(i, carry):
        hist_ref[pl.ds(i * _L, _L)] = jnp.zeros((_L,), jnp.int32)
        return carry

    lax.fori_loop(0, nbins // _L, body, 0)


def _process_row(xbuf, hist):
    ones = jnp.ones((_L,), jnp.int32)

    _zero_hist(hist, 2048)

    def p1(i, carry):
        key = _sortable_key(xbuf[pl.ds(i * _L, _L)])
        digit = (key >> jnp.uint32(21)).astype(jnp.int32)
        plsc.addupdate_scatter(hist, [digit], ones)
        return carry

    lax.fori_loop(0, _NCHUNK, p1, 0)
    b1, cgt1 = _scan_hist(hist, 2048, jnp.int32(_K))
    k2 = jnp.int32(_K) - cgt1
    b1u = b1.astype(jnp.uint32)

    _zero_hist(hist, 2048)

    def p2(i, carry):
        key = _sortable_key(xbuf[pl.ds(i * _L, _L)])
        m = (key >> jnp.uint32(21)) == b1u
        digit = ((key >> jnp.uint32(10)) & jnp.uint32(0x7FF)).astype(jnp.int32)
        plsc.addupdate_scatter(hist, [digit], ones, mask=m)
        return carry

    lax.fori_loop(0, _NCHUNK, p2, 0)
    b2, cgt2 = _scan_hist(hist, 2048, k2)
    k3 = k2 - cgt2
    pfx2 = (b1u << jnp.uint32(11)) | b2.astype(jnp.uint32)

    _zero_hist(hist, 1024)

    def p3(i, carry):
        key = _sortable_key(xbuf[pl.ds(i * _L, _L)])
        m = (key >> jnp.uint32(10)) == pfx2
        digit = (key & jnp.uint32(0x3FF)).astype(jnp.int32)
        plsc.addupdate_scatter(hist, [digit], ones, mask=m)
        return carry

    lax.fori_loop(0, _NCHUNK, p3, 0)
    b3, _ = _scan_hist(hist, 1024, k3)

    tkey = (pfx2 << jnp.uint32(10)) | b3.astype(jnp.uint32)

    tvec = jnp.full((_L,), tkey, jnp.uint32)

    def pm(i, carry):
        xv = xbuf[pl.ds(i * _L, _L)]
        key = _sortable_key(xv)
        xbuf[pl.ds(i * _L, _L)] = jnp.where(key >= tvec, xv, jnp.float32(0.0))
        return carry

    lax.fori_loop(0, _NCHUNK, pm, 0)


def _sc_body(x_hbm, out_hbm, xbuf, hist):
    wid = lax.axis_index("s") * 2 + lax.axis_index("c")
    for r in range(4):
        row = wid * 4 + r
        pltpu.sync_copy(x_hbm.at[row], xbuf)
        _process_row(xbuf, hist)
        pltpu.sync_copy(xbuf, out_hbm.at[row])


def kernel(x):
    mesh = plsc.VectorSubcoreMesh(core_axis_name="c", subcore_axis_name="s")
    f = functools.partial(
        pl.kernel,
        out_type=jax.ShapeDtypeStruct((_ROWS, _COLS), jnp.float32),
        mesh=mesh,
        scratch_types=[
            pltpu.VMEM((_COLS,), jnp.float32),
            pltpu.VMEM((2048,), jnp.int32),
        ],
        compiler_params=pltpu.CompilerParams(needs_layout_passes=False),
    )(_sc_body)
    return f(x)


# speedup vs baseline: 3.3671x; 3.3671x over previous
import functools

import jax
import jax.numpy as jnp
from jax import lax
from jax.experimental import pallas as pl
from jax.experimental.pallas import tpu as pltpu
from jax.experimental.pallas import tpu_sc as plsc

_K = 512
_COLS = 32768
_ROWS = 128
_L = 16
_NCHUNK = _COLS // _L


def _sortable_key(xv):
    u = lax.bitcast_convert_type(xv, jnp.uint32)
    neg = u >> jnp.uint32(31)
    return u ^ (neg * jnp.uint32(0x7FFFFFFF) + jnp.uint32(0x80000000))


def _scan_hist(hist_ref, nbins, k_needed):
    nch = nbins // _L

    def body(i, carry):
        cum, found, b_sel, cgt_sel = carry
        j = nch - 1 - i
        chunk = hist_ref[pl.ds(j * _L, _L)]
        rchunk = lax.rev(chunk, (0,))
        rcs = jnp.cumsum(rchunk)
        rc = lax.rev(rcs, (0,))
        cge = rc + cum
        cond = cge >= k_needed
        npos = jnp.sum(cond.astype(jnp.int32), axis=0)
        has = npos > 0
        b_cand = _L * j + npos - 1
        cgt_in = jnp.max(jnp.where(cond, jnp.int32(0), cge), axis=0)
        cgt_cand = jnp.where(npos == _L, cum, cgt_in)
        take = jnp.logical_and(jnp.logical_not(found), has)
        b_new = jnp.where(take, b_cand, b_sel)
        cgt_new = jnp.where(take, cgt_cand, cgt_sel)
        cum_new = cum + rcs[_L - 1]
        return cum_new, jnp.logical_or(found, has), b_new, cgt_new

    init = (jnp.int32(0), jnp.bool_(False), jnp.int32(0), jnp.int32(0))
    _, _, b_sel, cgt_sel = lax.fori_loop(0, nch, body, init)
    return b_sel, cgt_sel


def _zero_hist(hist):
    @plsc.parallel_loop(0, 256 // _L, unroll=4)
    def _(i):
        hist[pl.ds(i * _L, _L)] = jnp.zeros((_L,), jnp.int32)


def _process_row(xbuf, cand, hist):
    ones = jnp.ones((_L,), jnp.int32)
    iota = lax.iota(jnp.int32, _L)

    _zero_hist(hist)

    @plsc.parallel_loop(0, _NCHUNK, unroll=8)
    def _(i):
        key = _sortable_key(xbuf[pl.ds(i * _L, _L)])
        d1 = (key >> jnp.uint32(24)).astype(jnp.int32)
        plsc.addupdate_scatter(hist, [d1], ones)

    b1, cgt1 = _scan_hist(hist, 256, jnp.int32(_K))
    k2 = jnp.int32(_K) - cgt1
    b1u = b1.astype(jnp.uint32)

    _zero_hist(hist)

    @plsc.parallel_loop(0, _NCHUNK, unroll=4, carry=jnp.int32(0))
    def n1(i, off):
        key = _sortable_key(xbuf[pl.ds(i * _L, _L)])
        m = (key >> jnp.uint32(24)) == b1u
        d2 = ((key >> jnp.uint32(16)) & jnp.uint32(0xFF)).astype(jnp.int32)
        plsc.addupdate_scatter(hist, [d2], ones, mask=m)
        plsc.store_compressed(cand.at[pl.ds(off, _L)], key, mask=m)
        return off + jnp.sum(m.astype(jnp.int32), axis=0)

    b2, cgt2 = _scan_hist(hist, 256, k2)
    k3 = k2 - cgt2
    pfx2 = (b1u << jnp.uint32(8)) | b2.astype(jnp.uint32)

    _zero_hist(hist)
    nv1 = (n1 + (_L - 1)) // _L

    def p3(i, carry):
        key = cand[pl.ds(i * _L, _L)]
        valid = (i * _L + iota) < n1
        m = jnp.logical_and(valid, (key >> jnp.uint32(16)) == pfx2)
        d3 = ((key >> jnp.uint32(8)) & jnp.uint32(0xFF)).astype(jnp.int32)
        plsc.addupdate_scatter(hist, [d3], ones, mask=m)
        return carry

    lax.fori_loop(0, nv1, p3, 0)
    b3, cgt3 = _scan_hist(hist, 256, k3)
    k4 = k3 - cgt3
    pfx3 = (pfx2 << jnp.uint32(8)) | b3.astype(jnp.uint32)

    _zero_hist(hist)

    def p4(i, carry):
        key = cand[pl.ds(i * _L, _L)]
        valid = (i * _L + iota) < n1
        m = jnp.logical_and(valid, (key >> jnp.uint32(8)) == pfx3)
        d4 = (key & jnp.uint32(0xFF)).astype(jnp.int32)
        plsc.addupdate_scatter(hist, [d4], ones, mask=m)
        return carry

    lax.fori_loop(0, nv1, p4, 0)
    b4, _ = _scan_hist(hist, 256, k4)

    tkey = (pfx3 << jnp.uint32(8)) | b4.astype(jnp.uint32)
    tvec = jnp.full((_L,), tkey, jnp.uint32)

    @plsc.parallel_loop(0, _NCHUNK, unroll=8)
    def _(i):
        xv = xbuf[pl.ds(i * _L, _L)]
        key = _sortable_key(xv)
        xbuf[pl.ds(i * _L, _L)] = jnp.where(key >= tvec, xv, jnp.float32(0.0))


def _sc_body(x_hbm, out_hbm, xbuf, cand, hist):
    wid = lax.axis_index("s") * 2 + lax.axis_index("c")
    for r in range(4):
        row = wid * 4 + r
        pltpu.sync_copy(x_hbm.at[row], xbuf)
        _process_row(xbuf, cand, hist)
        pltpu.sync_copy(xbuf, out_hbm.at[row])


def kernel(x):
    mesh = plsc.VectorSubcoreMesh(core_axis_name="c", subcore_axis_name="s")
    f = functools.partial(
        pl.kernel,
        out_type=jax.ShapeDtypeStruct((_ROWS, _COLS), jnp.float32),
        mesh=mesh,
        scratch_types=[
            pltpu.VMEM((_COLS,), jnp.float32),
            pltpu.VMEM((_COLS + _L,), jnp.uint32),
            pltpu.VMEM((256,), jnp.int32),
        ],
        compiler_params=pltpu.CompilerParams(needs_layout_passes=False),
    )(_sc_body)
    return f(x)
